# baseline (speedup 1.0000x reference)
import functools

import jax
import jax.numpy as jnp
from jax import lax
from jax.experimental import pallas as pl
from jax.experimental.pallas import tpu as pltpu
from jax.experimental.pallas import tpu_sc as plsc

B, C, HW = 4, 384, 4096
K = HW // 2


_R, _C = 32, 128


def _bitonic_body(s_ref, o_ref):
    s = s_ref[...]
    bits = lax.bitcast_convert_type(s, jnp.int32)
    keys = jnp.where(bits >= 0, bits, bits ^ jnp.int32(0x7FFFFFFF))
    rio = lax.broadcasted_iota(jnp.int32, (B, _R, _C), 1)
    cio = lax.broadcasted_iota(jnp.int32, (B, _R, _C), 2)
    ids = rio * _C + cio
    n = _R * _C
    k = 2
    while k <= n:
        j = k // 2
        while j >= 1:
            if j < _R:
                jbit = (rio & j) != 0
                pk = jnp.where(jbit, jnp.roll(keys, j, axis=1),
                               jnp.roll(keys, -j, axis=1))
                pid = jnp.where(jbit, jnp.roll(ids, j, axis=1),
                                jnp.roll(ids, -j, axis=1))
            else:
                jc = j // _R
                jbit = (cio & jc) != 0
                pk = jnp.where(jbit, jnp.roll(keys, jc, axis=2),
                               jnp.roll(keys, -jc, axis=2))
                pid = jnp.where(jbit, jnp.roll(ids, jc, axis=2),
                                jnp.roll(ids, -jc, axis=2))
            if k < _R:
                up = (rio & k) == 0
            elif k < n:
                up = (cio & (k // _R)) == 0
            else:
                up = jnp.full((B, _R, _C), True)
            desired_min = jnp.logical_not(jbit) == up
            a_gt_b = (keys > pk) | ((keys == pk) & (ids > pid))
            tp = a_gt_b == desired_min
            keys = jnp.where(tp, pk, keys)
            ids = jnp.where(tp, pid, ids)
            j //= 2
        k *= 2
    o_ref[...] = jnp.swapaxes(ids[:, :, :_C // 2], 1, 2)


def _bitonic_call(scores):
    return pl.pallas_call(
        _bitonic_body,
        out_shape=jax.ShapeDtypeStruct((B, _C // 2, _R), jnp.int32),
    )(scores.reshape(B, _R, _C))


_ICHUNK = 128
_JCHUNK = 512


def _ranks_body(s_ref, c_ref, o_ref):
    ic = pl.program_id(1)
    col = c_ref[0]
    i_glob = ic * _ICHUNK + lax.broadcasted_iota(jnp.int32, (_ICHUNK, 1), 0)
    acc = jnp.zeros((_ICHUNK, 1), jnp.float32)
    for j in range(HW // _JCHUNK):
        row = s_ref[0, 0:1, j * _JCHUNK:(j + 1) * _JCHUNK]
        jio = j * _JCHUNK + lax.broadcasted_iota(jnp.int32, (1, _JCHUNK), 1)
        lt = row < col
        eq = (row == col) & (jio < i_glob)
        acc = acc + jnp.sum(jnp.where(lt | eq, 1.0, 0.0), axis=1, keepdims=True)
    o_ref[0] = acc.astype(jnp.int32)


def _ranks_call(scores):
    return pl.pallas_call(
        _ranks_body,
        grid=(B, HW // _ICHUNK),
        in_specs=[
            pl.BlockSpec((1, 1, HW), lambda b, i: (b, 0, 0)),
            pl.BlockSpec((1, _ICHUNK, 1), lambda b, i: (b, i, 0)),
        ],
        out_specs=pl.BlockSpec((1, _ICHUNK, 1), lambda b, i: (b, i, 0)),
        out_shape=jax.ShapeDtypeStruct((B, HW, 1), jnp.int32),
    )(scores.reshape(B, 1, HW), scores.reshape(B, HW, 1))


_SEGS = 8
_SEGW = K // _SEGS


def _scatter_sc(ranks_flat):
    mesh = plsc.VectorSubcoreMesh(core_axis_name="c", subcore_axis_name="s")

    @functools.partial(
        pl.kernel,
        out_type=jax.ShapeDtypeStruct((B * K,), jnp.int32),
        mesh=mesh,
        scratch_types=[
            pltpu.VMEM((HW,), jnp.int32),
            pltpu.VMEM((_SEGW,), jnp.int32),
        ],
        compiler_params=pltpu.CompilerParams(needs_layout_passes=False),
    )
    def k(ranks_hbm, out_hbm, ranks_v, out_v):
        wid = lax.axis_index("s") * 2 + lax.axis_index("c")
        batch = wid // _SEGS
        lo = (wid % _SEGS) * _SEGW
        pltpu.sync_copy(ranks_hbm.at[pl.ds(batch * HW, HW)], ranks_v)

        def body(kk, carry):
            rvec = ranks_v[pl.ds(kk * 16, 16)]
            ivec = kk * 16 + lax.iota(jnp.int32, 16)
            mask = (rvec >= lo) & (rvec < lo + _SEGW)
            local = jnp.where(mask, rvec - lo, 0)
            plsc.store_scatter(out_v, [local], ivec, mask=mask)
            return carry

        lax.fori_loop(0, HW // 16, body, 0)
        pltpu.sync_copy(out_v, out_hbm.at[pl.ds(batch * K + lo, _SEGW)])

    return k(ranks_flat)


def kernel(x, w_pred, b_pred):
    residual, _ = jnp.split(x, 2, axis=1)
    residual = residual ** 2
    simple_pred = jnp.einsum('bchw,c->bhw', residual, w_pred) + b_pred[0]
    simple_pred = jax.nn.sigmoid(simple_pred)[:, None, :, :]
    filter_scores = simple_pred + jnp.mean(residual, axis=1, keepdims=True)
    sorted_ids = _bitonic_call(filter_scores.reshape(B, HW))
    return filter_scores, sorted_ids.reshape(B, K)


# speedup vs baseline: 1.0165x; 1.0165x over previous
import functools

import jax
import jax.numpy as jnp
from jax import lax
from jax.experimental import pallas as pl
from jax.experimental.pallas import tpu as pltpu
from jax.experimental.pallas import tpu_sc as plsc

B, C, HW = 4, 384, 4096
K = HW // 2


_R, _C = 32, 128


def _bitonic_body(s_ref, o_ref):
    nb = s_ref.shape[0]
    s = s_ref[...]
    bits = lax.bitcast_convert_type(s, jnp.int32)
    keys = jnp.where(bits >= 0, bits, bits ^ jnp.int32(0x7FFFFFFF))
    rio = lax.broadcasted_iota(jnp.int32, (nb, _R, _C), 1)
    cio = lax.broadcasted_iota(jnp.int32, (nb, _R, _C), 2)
    ids = rio * _C + cio
    n = _R * _C
    k = 2
    while k <= n:
        j = k // 2
        while j >= 1:
            if j < _R:
                jbit = (rio & j) != 0
                pk = jnp.where(jbit, jnp.roll(keys, j, axis=1),
                               jnp.roll(keys, -j, axis=1))
                pid = jnp.where(jbit, jnp.roll(ids, j, axis=1),
                                jnp.roll(ids, -j, axis=1))
            else:
                jc = j // _R
                jbit = (cio & jc) != 0
                pk = jnp.where(jbit, jnp.roll(keys, jc, axis=2),
                               jnp.roll(keys, -jc, axis=2))
                pid = jnp.where(jbit, jnp.roll(ids, jc, axis=2),
                                jnp.roll(ids, -jc, axis=2))
            if k < _R:
                up = (rio & k) == 0
            elif k < n:
                up = (cio & (k // _R)) == 0
            else:
                up = jnp.full((nb, _R, _C), True)
            desired_min = jnp.logical_not(jbit) == up
            a_gt_b = (keys > pk) | ((keys == pk) & (ids > pid))
            tp = a_gt_b == desired_min
            kmin = jnp.minimum(keys, pk)
            kmax = jnp.maximum(keys, pk)
            keys = jnp.where(desired_min, kmin, kmax)
            ids = jnp.where(tp, pid, ids)
            j //= 2
        k *= 2
    o_ref[...] = jnp.swapaxes(ids[:, :, :_C // 2], 1, 2)


_GB = 4


def _bitonic_call(scores):
    return pl.pallas_call(
        _bitonic_body,
        grid=(B // _GB,),
        in_specs=[pl.BlockSpec((_GB, _R, _C), lambda g: (g, 0, 0))],
        out_specs=pl.BlockSpec((_GB, _C // 2, _R), lambda g: (g, 0, 0)),
        out_shape=jax.ShapeDtypeStruct((B, _C // 2, _R), jnp.int32),
    )(scores.reshape(B, _R, _C))


_ICHUNK = 128
_JCHUNK = 512


def _ranks_body(s_ref, c_ref, o_ref):
    ic = pl.program_id(1)
    col = c_ref[0]
    i_glob = ic * _ICHUNK + lax.broadcasted_iota(jnp.int32, (_ICHUNK, 1), 0)
    acc = jnp.zeros((_ICHUNK, 1), jnp.float32)
    for j in range(HW // _JCHUNK):
        row = s_ref[0, 0:1, j * _JCHUNK:(j + 1) * _JCHUNK]
        jio = j * _JCHUNK + lax.broadcasted_iota(jnp.int32, (1, _JCHUNK), 1)
        lt = row < col
        eq = (row == col) & (jio < i_glob)
        acc = acc + jnp.sum(jnp.where(lt | eq, 1.0, 0.0), axis=1, keepdims=True)
    o_ref[0] = acc.astype(jnp.int32)


def _ranks_call(scores):
    return pl.pallas_call(
        _ranks_body,
        grid=(B, HW // _ICHUNK),
        in_specs=[
            pl.BlockSpec((1, 1, HW), lambda b, i: (b, 0, 0)),
            pl.BlockSpec((1, _ICHUNK, 1), lambda b, i: (b, i, 0)),
        ],
        out_specs=pl.BlockSpec((1, _ICHUNK, 1), lambda b, i: (b, i, 0)),
        out_shape=jax.ShapeDtypeStruct((B, HW, 1), jnp.int32),
    )(scores.reshape(B, 1, HW), scores.reshape(B, HW, 1))


_SEGS = 8
_SEGW = K // _SEGS


def _scatter_sc(ranks_flat):
    mesh = plsc.VectorSubcoreMesh(core_axis_name="c", subcore_axis_name="s")

    @functools.partial(
        pl.kernel,
        out_type=jax.ShapeDtypeStruct((B * K,), jnp.int32),
        mesh=mesh,
        scratch_types=[
            pltpu.VMEM((HW,), jnp.int32),
            pltpu.VMEM((_SEGW,), jnp.int32),
        ],
        compiler_params=pltpu.CompilerParams(needs_layout_passes=False),
    )
    def k(ranks_hbm, out_hbm, ranks_v, out_v):
        wid = lax.axis_index("s") * 2 + lax.axis_index("c")
        batch = wid // _SEGS
        lo = (wid % _SEGS) * _SEGW
        pltpu.sync_copy(ranks_hbm.at[pl.ds(batch * HW, HW)], ranks_v)

        def body(kk, carry):
            rvec = ranks_v[pl.ds(kk * 16, 16)]
            ivec = kk * 16 + lax.iota(jnp.int32, 16)
            mask = (rvec >= lo) & (rvec < lo + _SEGW)
            local = jnp.where(mask, rvec - lo, 0)
            plsc.store_scatter(out_v, [local], ivec, mask=mask)
            return carry

        lax.fori_loop(0, HW // 16, body, 0)
        pltpu.sync_copy(out_v, out_hbm.at[pl.ds(batch * K + lo, _SEGW)])

    return k(ranks_flat)


def kernel(x, w_pred, b_pred):
    residual, _ = jnp.split(x, 2, axis=1)
    residual = residual ** 2
    simple_pred = jnp.einsum('bchw,c->bhw', residual, w_pred) + b_pred[0]
    simple_pred = jax.nn.sigmoid(simple_pred)[:, None, :, :]
    filter_scores = simple_pred + jnp.mean(residual, axis=1, keepdims=True)
    sorted_ids = _bitonic_call(filter_scores.reshape(B, HW))
    return filter_scores, sorted_ids.reshape(B, K)


# speedup vs baseline: 1.0198x; 1.0032x over previous
import functools

import jax
import jax.numpy as jnp
from jax import lax
from jax.experimental import pallas as pl
from jax.experimental.pallas import tpu as pltpu
from jax.experimental.pallas import tpu_sc as plsc

B, C, HW = 4, 384, 4096
K = HW // 2


_R, _C = 32, 128


def _bitonic_body(s_ref, o_ref):
    nb = s_ref.shape[0]
    s = s_ref[...]
    bits = lax.bitcast_convert_type(s, jnp.int32)
    keys = jnp.where(bits >= 0, bits, bits ^ jnp.int32(0x7FFFFFFF))
    rio = lax.broadcasted_iota(jnp.int32, (nb, _R, _C), 1)
    cio = lax.broadcasted_iota(jnp.int32, (nb, _R, _C), 2)
    ids = rio * _C + cio
    n = _R * _C
    k = 2
    while k <= n:
        j = k // 2
        while j >= 1:
            if 8 <= j < _R:
                jbit = (rio & j) != 0
                nbk = _R // j
                perm = [p ^ 1 for p in range(nbk)]
                pk = jnp.concatenate(
                    [keys[:, j * p:j * (p + 1)] for p in perm], axis=1)
                pid = jnp.concatenate(
                    [ids[:, j * p:j * (p + 1)] for p in perm], axis=1)
            elif j < _R:
                jbit = (rio & j) != 0
                pk = jnp.where(jbit, jnp.roll(keys, j, axis=1),
                               jnp.roll(keys, -j, axis=1))
                pid = jnp.where(jbit, jnp.roll(ids, j, axis=1),
                                jnp.roll(ids, -j, axis=1))
            else:
                jc = j // _R
                jbit = (cio & jc) != 0
                pk = jnp.where(jbit, jnp.roll(keys, jc, axis=2),
                               jnp.roll(keys, -jc, axis=2))
                pid = jnp.where(jbit, jnp.roll(ids, jc, axis=2),
                                jnp.roll(ids, -jc, axis=2))
            if k < _R:
                up = (rio & k) == 0
            elif k < n:
                up = (cio & (k // _R)) == 0
            else:
                up = jnp.full((nb, _R, _C), True)
            desired_min = jnp.logical_not(jbit) == up
            a_gt_b = (keys > pk) | ((keys == pk) & (ids > pid))
            tp = a_gt_b == desired_min
            kmin = jnp.minimum(keys, pk)
            kmax = jnp.maximum(keys, pk)
            keys = jnp.where(desired_min, kmin, kmax)
            ids = jnp.where(tp, pid, ids)
            j //= 2
        k *= 2
    o_ref[...] = jnp.swapaxes(ids[:, :, :_C // 2], 1, 2)


_GB = 4


def _bitonic_call(scores):
    return pl.pallas_call(
        _bitonic_body,
        grid=(B // _GB,),
        in_specs=[pl.BlockSpec((_GB, _R, _C), lambda g: (g, 0, 0))],
        out_specs=pl.BlockSpec((_GB, _C // 2, _R), lambda g: (g, 0, 0)),
        out_shape=jax.ShapeDtypeStruct((B, _C // 2, _R), jnp.int32),
    )(scores.reshape(B, _R, _C))


_ICHUNK = 128
_JCHUNK = 512


def _ranks_body(s_ref, c_ref, o_ref):
    ic = pl.program_id(1)
    col = c_ref[0]
    i_glob = ic * _ICHUNK + lax.broadcasted_iota(jnp.int32, (_ICHUNK, 1), 0)
    acc = jnp.zeros((_ICHUNK, 1), jnp.float32)
    for j in range(HW // _JCHUNK):
        row = s_ref[0, 0:1, j * _JCHUNK:(j + 1) * _JCHUNK]
        jio = j * _JCHUNK + lax.broadcasted_iota(jnp.int32, (1, _JCHUNK), 1)
        lt = row < col
        eq = (row == col) & (jio < i_glob)
        acc = acc + jnp.sum(jnp.where(lt | eq, 1.0, 0.0), axis=1, keepdims=True)
    o_ref[0] = acc.astype(jnp.int32)


def _ranks_call(scores):
    return pl.pallas_call(
        _ranks_body,
        grid=(B, HW // _ICHUNK),
        in_specs=[
            pl.BlockSpec((1, 1, HW), lambda b, i: (b, 0, 0)),
            pl.BlockSpec((1, _ICHUNK, 1), lambda b, i: (b, i, 0)),
        ],
        out_specs=pl.BlockSpec((1, _ICHUNK, 1), lambda b, i: (b, i, 0)),
        out_shape=jax.ShapeDtypeStruct((B, HW, 1), jnp.int32),
    )(scores.reshape(B, 1, HW), scores.reshape(B, HW, 1))


_SEGS = 8
_SEGW = K // _SEGS


def _scatter_sc(ranks_flat):
    mesh = plsc.VectorSubcoreMesh(core_axis_name="c", subcore_axis_name="s")

    @functools.partial(
        pl.kernel,
        out_type=jax.ShapeDtypeStruct((B * K,), jnp.int32),
        mesh=mesh,
        scratch_types=[
            pltpu.VMEM((HW,), jnp.int32),
            pltpu.VMEM((_SEGW,), jnp.int32),
        ],
        compiler_params=pltpu.CompilerParams(needs_layout_passes=False),
    )
    def k(ranks_hbm, out_hbm, ranks_v, out_v):
        wid = lax.axis_index("s") * 2 + lax.axis_index("c")
        batch = wid // _SEGS
        lo = (wid % _SEGS) * _SEGW
        pltpu.sync_copy(ranks_hbm.at[pl.ds(batch * HW, HW)], ranks_v)

        def body(kk, carry):
            rvec = ranks_v[pl.ds(kk * 16, 16)]
            ivec = kk * 16 + lax.iota(jnp.int32, 16)
            mask = (rvec >= lo) & (rvec < lo + _SEGW)
            local = jnp.where(mask, rvec - lo, 0)
            plsc.store_scatter(out_v, [local], ivec, mask=mask)
            return carry

        lax.fori_loop(0, HW // 16, body, 0)
        pltpu.sync_copy(out_v, out_hbm.at[pl.ds(batch * K + lo, _SEGW)])

    return k(ranks_flat)


def kernel(x, w_pred, b_pred):
    residual, _ = jnp.split(x, 2, axis=1)
    residual = residual ** 2
    simple_pred = jnp.einsum('bchw,c->bhw', residual, w_pred) + b_pred[0]
    simple_pred = jax.nn.sigmoid(simple_pred)[:, None, :, :]
    filter_scores = simple_pred + jnp.mean(residual, axis=1, keepdims=True)
    sorted_ids = _bitonic_call(filter_scores.reshape(B, HW))
    return filter_scores, sorted_ids.reshape(B, K)


# speedup vs baseline: 1.0198x; 1.0000x over previous
import jax
import jax.numpy as jnp
from jax import lax
from jax.experimental import pallas as pl

B, HW = 4, 4096
K = HW // 2


_R, _C = 32, 128


def _bitonic_body(s_ref, o_ref):
    nb = s_ref.shape[0]
    s = s_ref[...]
    bits = lax.bitcast_convert_type(s, jnp.int32)
    keys = jnp.where(bits >= 0, bits, bits ^ jnp.int32(0x7FFFFFFF))
    rio = lax.broadcasted_iota(jnp.int32, (nb, _R, _C), 1)
    cio = lax.broadcasted_iota(jnp.int32, (nb, _R, _C), 2)
    ids = rio * _C + cio
    n = _R * _C
    k = 2
    while k <= n:
        j = k // 2
        while j >= 1:
            if 8 <= j < _R:
                jbit = (rio & j) != 0
                nbk = _R // j
                perm = [p ^ 1 for p in range(nbk)]
                pk = jnp.concatenate(
                    [keys[:, j * p:j * (p + 1)] for p in perm], axis=1)
                pid = jnp.concatenate(
                    [ids[:, j * p:j * (p + 1)] for p in perm], axis=1)
            elif j < _R:
                jbit = (rio & j) != 0
                pk = jnp.where(jbit, jnp.roll(keys, j, axis=1),
                               jnp.roll(keys, -j, axis=1))
                pid = jnp.where(jbit, jnp.roll(ids, j, axis=1),
                                jnp.roll(ids, -j, axis=1))
            else:
                jc = j // _R
                jbit = (cio & jc) != 0
                pk = jnp.where(jbit, jnp.roll(keys, jc, axis=2),
                               jnp.roll(keys, -jc, axis=2))
                pid = jnp.where(jbit, jnp.roll(ids, jc, axis=2),
                                jnp.roll(ids, -jc, axis=2))
            if k < _R:
                up = (rio & k) == 0
            elif k < n:
                up = (cio & (k // _R)) == 0
            else:
                up = jnp.full((nb, _R, _C), True)
            desired_min = jnp.logical_not(jbit) == up
            a_gt_b = (keys > pk) | ((keys == pk) & (ids > pid))
            tp = a_gt_b == desired_min
            kmin = jnp.minimum(keys, pk)
            kmax = jnp.maximum(keys, pk)
            keys = jnp.where(desired_min, kmin, kmax)
            ids = jnp.where(tp, pid, ids)
            j //= 2
        k *= 2
    o_ref[...] = jnp.swapaxes(ids[:, :, :_C // 2], 1, 2)


_GB = 4


def _bitonic_call(scores):
    return pl.pallas_call(
        _bitonic_body,
        grid=(B // _GB,),
        in_specs=[pl.BlockSpec((_GB, _R, _C), lambda g: (g, 0, 0))],
        out_specs=pl.BlockSpec((_GB, _C // 2, _R), lambda g: (g, 0, 0)),
        out_shape=jax.ShapeDtypeStruct((B, _C // 2, _R), jnp.int32),
    )(scores.reshape(B, _R, _C))


def kernel(x, w_pred, b_pred):
    residual, _ = jnp.split(x, 2, axis=1)
    residual = residual ** 2
    simple_pred = jnp.einsum('bchw,c->bhw', residual, w_pred) + b_pred[0]
    simple_pred = jax.nn.sigmoid(simple_pred)[:, None, :, :]
    filter_scores = simple_pred + jnp.mean(residual, axis=1, keepdims=True)
    sorted_ids = _bitonic_call(filter_scores.reshape(B, HW))
    return filter_scores, sorted_ids.reshape(B, K)
